# baseline (speedup 1.0000x reference)
import jax
import jax.numpy as jnp
from jax.experimental import pallas as pl

N_NODES = 10000
HID_C = 16
OUT_C = 8
HEADS1 = 8
HEADS2 = 1


def _gat(x, edge_index, edge_attr, W, att_src, att_dst, W_edge, att_edge, bias, heads, out_c):
    N = x.shape[0]
    src = edge_index[0]
    dst = edge_index[1]
    loop = jnp.arange(N, dtype=edge_index.dtype)
    src = jnp.concatenate([src, loop], axis=0)
    dst = jnp.concatenate([dst, loop], axis=0)
    mean_attr = jnp.mean(edge_attr, axis=0, keepdims=True)
    ea = jnp.concatenate([edge_attr, jnp.broadcast_to(mean_attr, (N, edge_attr.shape[1]))], axis=0)
    h = (x @ W).reshape(N, heads, out_c)
    a_src = jnp.sum(h * att_src, axis=-1)
    a_dst = jnp.sum(h * att_dst, axis=-1)
    e_h = (ea @ W_edge).reshape(-1, heads, out_c)
    a_edge = jnp.sum(e_h * att_edge, axis=-1)
    alpha = a_src[src] + a_dst[dst] + a_edge
    alpha = jax.nn.leaky_relu(alpha, negative_slope=0.2)
    m = jax.ops.segment_max(alpha, dst, num_segments=N)
    alpha = jnp.exp(alpha - m[dst])
    s = jax.ops.segment_sum(alpha, dst, num_segments=N)
    alpha = alpha / (s[dst] + 1e-16)
    msg = h[src] * alpha[:, :, None]
    out = jax.ops.segment_sum(msg, dst, num_segments=N)
    out = jnp.mean(out, axis=1) + bias
    return out


def _bias_relu_kernel(x_ref, o_ref):
    o_ref[...] = jnp.maximum(x_ref[...], 0.0)


def kernel(x, edge_index, edge_attr,
           W1, att_src1, att_dst1, W_edge1, att_edge1, bias1,
           W2, att_src2, att_dst2, W_edge2, att_edge2, bias2):
    h = _gat(x, edge_index, edge_attr, W1, att_src1, att_dst1, W_edge1, att_edge1, bias1, HEADS1, HID_C)
    h = pl.pallas_call(
        _bias_relu_kernel,
        out_shape=jax.ShapeDtypeStruct(h.shape, h.dtype),
    )(h)
    out = _gat(h, edge_index, edge_attr, W2, att_src2, att_dst2, W_edge2, att_edge2, bias2, HEADS2, OUT_C)
    return out


# speedup vs baseline: 82.1360x; 82.1360x over previous
import functools

import jax
import jax.numpy as jnp
import numpy as np
from jax import lax
from jax.experimental import pallas as pl
from jax.experimental.pallas import tpu as pltpu
from jax.experimental.pallas import tpu_sc as plsc

N = 10000
E = 320000
IN_C = 128
HID_C = 16
OUT_C = 8
H1 = 8
HC = H1 * HID_C
ROWW = HC + 16
ROWW2 = 16

NCORES = 2
NSUB = 16
NTILES = NCORES * NSUB
EPT = E // NTILES
CHUNK = 80
NCHUNK = EPT // CHUNK
NPAD = 10240
NPT = NPAD // NSUB
ZROWS = 128

_f32 = jnp.float32
_i32 = jnp.int32

_S_BLK = np.repeat(np.eye(H1, dtype=np.float32), HID_C, axis=0)
_E8 = _S_BLK.T.copy()
_M16 = np.zeros((HC, HID_C), dtype=np.float32)
for _j in range(HC):
    _M16[_j, _j % HID_C] = 1.0 / H1
_ONEHOT16 = np.eye(16, dtype=np.float32)


def _isplat(i):
    return lax.iota(_i32, 16) * 0 + i


def _zeros16():
    return lax.broadcast_in_dim(_f32(0.0), (16,), ())


def _splat(vec, i):
    idx = _isplat(i)
    return lax.gather(
        vec, idx[:, None],
        lax.GatherDimensionNumbers(offset_dims=(), collapsed_slice_dims=(0,),
                                   start_index_map=(0,)),
        (1,), mode=lax.GatherScatterMode.PROMISE_IN_BOUNDS)


def _lrelu_exp(z):
    return jnp.exp(jnp.maximum(z, 0.2 * z))


def _prep1_body(x_ref, w1_ref, asrcm_ref, adstm_ref, ear_ref,
                g1_ref, asrct_ref, adstt_ref, mean_ref):
    h1 = jnp.dot(x_ref[...], w1_ref[...], preferred_element_type=_f32)
    a_src = jnp.dot(h1, asrcm_ref[...], preferred_element_type=_f32)
    a_dst = jnp.dot(h1, adstm_ref[...], preferred_element_type=_f32)
    z8 = jnp.zeros((N, 8), _f32)
    g1_ref[...] = h1
    asrct_ref[...] = jnp.concatenate([a_src, z8], axis=1)
    adstt_ref[...] = jnp.concatenate([a_dst, z8], axis=1)
    mean_ref[...] = (jnp.sum(ear_ref[...]) / E).reshape(1, 1)


BR = 2000


def _post1_body(accm_ref, accw_ref, g1_ref, asrct_ref, adstt_ref,
                mc1_ref, bias1_ref,
                e8_ref, m16_ref, w2_ref, asrc2m_ref, adst2m_ref, mc2_ref,
                g2_ref, adst2t_ref, wself2_ref):
    msum = accm_ref[0] + accm_ref[1]
    wsum = accw_ref[0] + accw_ref[1]
    h1 = g1_ref[...]
    a_src = asrct_ref[:, :8]
    a_dst = adstt_ref[:, :8]
    wself = _lrelu_exp(a_src + a_dst + mc1_ref[...])
    w128 = jnp.dot(wself, e8_ref[...], preferred_element_type=_f32)
    s128 = jnp.dot(wsum[:, :8] + wself, e8_ref[...],
                   preferred_element_type=_f32)
    msg = msum + w128 * h1
    pre = msg / s128
    out1 = jnp.maximum(
        jnp.dot(pre, m16_ref[...], preferred_element_type=_f32) + bias1_ref[...],
        0.0)
    h2 = jnp.dot(out1, w2_ref[...], preferred_element_type=_f32)
    a2s = jnp.dot(h2, asrc2m_ref[...], preferred_element_type=_f32)
    a2d = jnp.dot(h2, adst2m_ref[...], preferred_element_type=_f32)
    z2 = a2s + a2d + mc2_ref[...]
    wself2 = _lrelu_exp(z2)
    ones1 = jnp.ones((BR, 1), _f32)
    g2_ref[...] = jnp.concatenate(
        [h2, ones1, a2s[:, 0:1], jnp.zeros((BR, 6), _f32)], axis=1)
    adst2t_ref[...] = jnp.broadcast_to(a2d[:, 0:1], (BR, 16))
    wself2_ref[...] = jnp.broadcast_to(wself2[:, 0:1], (BR, 16))


def _post2_body(acc2_ref, g2_ref, wself2_ref, bias2_ref, out_ref):
    tot = acc2_ref[0, :N] + acc2_ref[1, :N] + wself2_ref[...] * g2_ref[...]
    s = tot[:, OUT_C:OUT_C + 1]
    out_ref[...] = tot[:, :OUT_C] / s + bias2_ref[...]


_MESH = plsc.VectorSubcoreMesh(core_axis_name="c", subcore_axis_name="s")


@functools.partial(
    pl.kernel,
    mesh=_MESH,
    compiler_params=pltpu.CompilerParams(use_tc_tiling_on_sc=False),
    out_type=(
        jax.ShapeDtypeStruct((NCORES, NPAD, HC), _f32),
        jax.ShapeDtypeStruct((NCORES, NPAD, 16), _f32),
    ),
    scratch_types=[
        pltpu.VMEM((CHUNK,), _i32),
        pltpu.VMEM((CHUNK,), _i32),
        pltpu.VMEM((CHUNK,), _f32),
        pltpu.VMEM((CHUNK, HC), _f32),
        pltpu.VMEM((CHUNK, 16), _f32),
        pltpu.VMEM((CHUNK, 16), _f32),
        pltpu.VMEM((CHUNK, 16), _f32),
        pltpu.VMEM((16,), _f32),
        pltpu.VMEM((ZROWS, HC), _f32),
        pltpu.VMEM((ZROWS, 16), _f32),
        pltpu.VMEM_SHARED((NPAD, HC), _f32),
        pltpu.VMEM_SHARED((NPAD, 16), _f32),
        pltpu.SemaphoreType.DMA,
        pltpu.SemaphoreType.DMA,
        pltpu.SemaphoreType.DMA,
        pltpu.SemaphoreType.DMA,
        pltpu.SemaphoreType.DMA,
        pltpu.SemaphoreType.DMA,
    ],
)
def _edges1(g1, asrct, adstt, src_a, dst_a, ea_a, c1_a, outm, outw,
            srcv, dstv, eav, rows, asrcr, adrows, wbuf, c1v, zbuf, zbuf2,
            accm, accw, sem1, sem2, sem3, sem4, sem5, sem6):
    cc = lax.axis_index("c")
    ss = lax.axis_index("s")

    def zrow(i, carry):
        for j in range(HC // 16):
            zbuf[i, pl.ds(16 * j, 16)] = _zeros16()
        zbuf2[i, :] = _zeros16()
        return carry
    lax.fori_loop(0, ZROWS, zrow, 0)
    row0 = ss * NPT
    for k in range(NPT // ZROWS):
        pltpu.sync_copy(zbuf, accm.at[pl.ds(row0 + ZROWS * k, ZROWS)])
        pltpu.sync_copy(zbuf2, accw.at[pl.ds(row0 + ZROWS * k, ZROWS)])
    pltpu.sync_copy(c1_a, c1v)
    plsc.subcore_barrier()

    c1vec = c1v[...]
    ebase = cc * (E // NCORES) + ss * EPT

    def chunk_body(t, carry):
        cb = ebase + t * CHUNK
        s_cp = pltpu.async_copy(src_a.at[pl.ds(cb, CHUNK)], srcv, sem1)
        d_cp = pltpu.async_copy(dst_a.at[pl.ds(cb, CHUNK)], dstv, sem2)
        e_cp = pltpu.async_copy(ea_a.at[pl.ds(cb, CHUNK)], eav, sem3)
        s_cp.wait()
        g_cp = pltpu.async_copy(g1.at[srcv], rows, sem4)
        as_cp = pltpu.async_copy(asrct.at[srcv], asrcr, sem5)
        d_cp.wait()
        a_cp = pltpu.async_copy(adstt.at[dstv], adrows, sem6)
        e_cp.wait()
        g_cp.wait()
        as_cp.wait()
        a_cp.wait()

        for g in range(CHUNK // 16):
            ea16 = eav[pl.ds(16 * g, 16)]
            for e in range(16):
                ei = 16 * g + e
                w = _lrelu_exp(asrcr[ei, :] + adrows[ei, :]
                               + _splat(ea16, e) * c1vec)
                wbuf[ei, :] = w
                for h in range(H1):
                    wh = _splat(w, h)
                    rows[ei, pl.ds(16 * h, 16)] = rows[ei, pl.ds(16 * h, 16)] * wh
        pltpu.sync_copy(rows, accm.at[dstv], add=True)
        pltpu.sync_copy(wbuf, accw.at[dstv], add=True)
        return carry
    lax.fori_loop(0, NCHUNK, chunk_body, 0)

    plsc.subcore_barrier()
    pltpu.sync_copy(accm.at[pl.ds(row0, NPT)], outm.at[cc, pl.ds(row0, NPT)])
    pltpu.sync_copy(accw.at[pl.ds(row0, NPT)], outw.at[cc, pl.ds(row0, NPT)])


@functools.partial(
    pl.kernel,
    mesh=_MESH,
    compiler_params=pltpu.CompilerParams(use_tc_tiling_on_sc=False),
    out_type=jax.ShapeDtypeStruct((NCORES, NPAD, ROWW2), _f32),
    scratch_types=[
        pltpu.VMEM((CHUNK,), _i32),
        pltpu.VMEM((CHUNK,), _i32),
        pltpu.VMEM((CHUNK,), _f32),
        pltpu.VMEM((CHUNK, ROWW2), _f32),
        pltpu.VMEM((CHUNK, 16), _f32),
        pltpu.VMEM((16,), _f32),
        pltpu.VMEM((ZROWS, ROWW2), _f32),
        pltpu.VMEM_SHARED((NPAD, ROWW2), _f32),
        pltpu.SemaphoreType.DMA,
        pltpu.SemaphoreType.DMA,
        pltpu.SemaphoreType.DMA,
        pltpu.SemaphoreType.DMA,
        pltpu.SemaphoreType.DMA,
    ],
)
def _edges2(g2, adst2t, src_a, dst_a, ea_a, c2_a, out,
            srcv, dstv, eav, rows, adrows, c2v, zbuf, acc,
            sem1, sem2, sem3, sem4, sem5):
    cc = lax.axis_index("c")
    ss = lax.axis_index("s")

    def zrow(i, carry):
        zbuf[i, :] = _zeros16()
        return carry
    lax.fori_loop(0, ZROWS, zrow, 0)
    row0 = ss * NPT
    for k in range(NPT // ZROWS):
        pltpu.sync_copy(zbuf, acc.at[pl.ds(row0 + ZROWS * k, ZROWS)])
    pltpu.sync_copy(c2_a, c2v)
    plsc.subcore_barrier()

    c2vec = c2v[...]
    ebase = cc * (E // NCORES) + ss * EPT

    def chunk_body(t, carry):
        cb = ebase + t * CHUNK
        s_cp = pltpu.async_copy(src_a.at[pl.ds(cb, CHUNK)], srcv, sem1)
        d_cp = pltpu.async_copy(dst_a.at[pl.ds(cb, CHUNK)], dstv, sem2)
        e_cp = pltpu.async_copy(ea_a.at[pl.ds(cb, CHUNK)], eav, sem3)
        s_cp.wait()
        g_cp = pltpu.async_copy(g2.at[srcv], rows, sem4)
        d_cp.wait()
        a_cp = pltpu.async_copy(adst2t.at[dstv], adrows, sem5)
        e_cp.wait()
        g_cp.wait()
        a_cp.wait()

        for g in range(CHUNK // 16):
            ea16 = eav[pl.ds(16 * g, 16)]
            for e in range(16):
                ei = 16 * g + e
                r = rows[ei, :]
                z = _splat(r, 9) + adrows[ei, :] + _splat(ea16, e) * c2vec
                rows[ei, :] = r * _lrelu_exp(z)
        pltpu.sync_copy(rows, acc.at[dstv], add=True)
        return carry
    lax.fori_loop(0, NCHUNK, chunk_body, 0)

    plsc.subcore_barrier()
    pltpu.sync_copy(acc.at[pl.ds(row0, NPT)], out.at[cc, pl.ds(row0, NPT)])


def kernel(x, edge_index, edge_attr,
           W1, att_src1, att_dst1, W_edge1, att_edge1, bias1,
           W2, att_src2, att_dst2, W_edge2, att_edge2, bias2):
    src = edge_index[0].astype(_i32)
    dst = edge_index[1].astype(_i32)
    ea = edge_attr[:, 0]

    asrc1m = att_src1.reshape(HC, 1) * _S_BLK
    adst1m = att_dst1.reshape(HC, 1) * _S_BLK
    c1 = jnp.sum(W_edge1.reshape(H1, HID_C) * att_edge1[0], axis=-1)
    c1pad = jnp.concatenate([c1, jnp.zeros((8,), _f32)])
    c2 = jnp.sum(W_edge2.reshape(OUT_C) * att_edge2.reshape(OUT_C))
    c2vec = jnp.full((16,), c2, dtype=_f32)
    asrc2m = att_src2.reshape(OUT_C, 1) * np.eye(OUT_C, 16, dtype=np.float32)
    adst2m = att_dst2.reshape(OUT_C, 1) * np.eye(OUT_C, 16, dtype=np.float32)

    g1, asrct, adstt, mean_s = pl.pallas_call(
        _prep1_body,
        out_shape=(
            jax.ShapeDtypeStruct((N, HC), _f32),
            jax.ShapeDtypeStruct((N, 16), _f32),
            jax.ShapeDtypeStruct((N, 16), _f32),
            jax.ShapeDtypeStruct((1, 1), _f32),
        ),
    )(x, W1, asrc1m, adst1m, edge_attr.reshape(2500, 128))

    accm, accw = _edges1(g1, asrct, adstt, src, dst, ea, c1pad)

    mc1 = mean_s[0, 0] * c1.reshape(1, 8)
    mc2 = jnp.broadcast_to(mean_s[0, 0] * c2, (1, 16))

    _full = lambda bs: pl.BlockSpec(bs, lambda i: (0,) * len(bs))
    g2, adst2t, wself2 = pl.pallas_call(
        _post1_body,
        grid=(N // BR,),
        in_specs=[
            pl.BlockSpec((2, BR, HC), lambda i: (0, i, 0)),
            pl.BlockSpec((2, BR, 16), lambda i: (0, i, 0)),
            pl.BlockSpec((BR, HC), lambda i: (i, 0)),
            pl.BlockSpec((BR, 16), lambda i: (i, 0)),
            pl.BlockSpec((BR, 16), lambda i: (i, 0)),
            _full((1, 8)), _full((1, 16)), _full((8, HC)), _full((HC, 16)),
            _full((16, 8)), _full((8, 16)), _full((8, 16)), _full((1, 16)),
        ],
        out_specs=[pl.BlockSpec((BR, 16), lambda i: (i, 0))] * 3,
        out_shape=(
            jax.ShapeDtypeStruct((N, 16), _f32),
            jax.ShapeDtypeStruct((N, 16), _f32),
            jax.ShapeDtypeStruct((N, 16), _f32),
        ),
    )(accm, accw, g1, asrct, adstt, mc1, bias1.reshape(1, HID_C),
      jnp.asarray(_E8), jnp.asarray(_M16), W2, asrc2m, adst2m, mc2)

    acc2 = _edges2(g2, adst2t, src, dst, ea, c2vec)

    out = pl.pallas_call(
        _post2_body,
        out_shape=jax.ShapeDtypeStruct((N, OUT_C), _f32),
    )(acc2, g2, wself2, bias2.reshape(1, OUT_C))
    return out
